# baseline (speedup 1.0000x reference)
import functools

import jax
import jax.numpy as jnp
from jax import lax
from jax.experimental import pallas as pl
from jax.experimental.pallas import tpu as pltpu
from jax.experimental.pallas import tpu_sc as plsc

_SEQ = 1024
_DIM = 64
_ROWS = _SEQ * _SEQ
_NW = 32
_CHUNK = 128
_PER_W = _ROWS // _NW
_NCH = _PER_W // _CHUNK


def _sc_kernel(table_hbm, x_hbm, out_hbm, idx_v, xrows_v, trows_v, sem):
    wid = lax.axis_index("s") * 2 + lax.axis_index("c")
    base = wid * _PER_W

    def chunk_body(c, _):
        row0 = base + c * _CHUNK
        for v in range(_CHUNK // 16):
            b0 = jnp.full((16,), row0 + v * 16, jnp.int32)
            m = b0 + lax.iota(jnp.int32, 16)
            idx_v[pl.ds(v * 16, 16)] = (
                jnp.full((16,), _SEQ - 1, jnp.int32)
                + (m >> 10)
                - (m & (_SEQ - 1))
            )
        pltpu.sync_copy(x_hbm.at[pl.ds(row0 * _DIM, _CHUNK * _DIM)], xrows_v)
        pltpu.async_copy(table_hbm.at[idx_v], trows_v, sem).wait()

        def add_row(r, _):
            for l in range(_DIM // 16):
                sl = pl.ds(r * _DIM + l * 16, 16)
                xrows_v[sl] = xrows_v[sl] + trows_v[r, pl.ds(l * 16, 16)]
            return ()

        lax.fori_loop(0, _CHUNK, add_row, ())
        pltpu.sync_copy(xrows_v, out_hbm.at[pl.ds(row0 * _DIM, _CHUNK * _DIM)])
        return ()

    lax.fori_loop(0, _NCH, chunk_body, ())


def kernel(x, relative_embedding):
    x1 = x.reshape(_ROWS * _DIM)
    run = functools.partial(
        pl.kernel,
        mesh=plsc.VectorSubcoreMesh(core_axis_name="c", subcore_axis_name="s"),
        out_type=jax.ShapeDtypeStruct((_ROWS * _DIM,), jnp.float32),
        scratch_types=[
            pltpu.VMEM((_CHUNK,), jnp.int32),
            pltpu.VMEM((_CHUNK * _DIM,), jnp.float32),
            pltpu.VMEM((_CHUNK, 2 * _DIM), jnp.float32),
            pltpu.SemaphoreType.DMA,
        ],
    )(_sc_kernel)
    table2 = jnp.pad(relative_embedding, ((0, 0), (0, _DIM)))
    out = run(table2, x1)
    return out.reshape(x.shape)


# speedup vs baseline: 14.0455x; 14.0455x over previous
import jax
import jax.numpy as jnp
from jax.experimental import pallas as pl
from jax.experimental.pallas import tpu as pltpu

_SEQ = 1024
_DIM = 64
_BI = 32
_WIN = _SEQ + 128


def _body(table_ref, x_ref, o_ref):
    i0 = pl.program_id(0) * _BI
    for r in range(_BI):
        s = _SEQ - 1 - (i0 + r)
        a = pl.multiple_of((s // 128) * 128, 128)
        b = s - a
        coarse = table_ref[:, pl.ds(a, _WIN)]
        win = pltpu.roll(coarse, (_WIN - b) % _WIN, axis=1)
        o_ref[0, r] = x_ref[0, r] + win[:, :_SEQ]


def kernel(x, relative_embedding):
    rt_t = relative_embedding[::-1].T
    rt_p = jnp.pad(rt_t, ((0, 0), (0, 1)))
    xt = jnp.transpose(x, (0, 1, 3, 2))
    out = pl.pallas_call(
        _body,
        grid=(_SEQ // _BI,),
        in_specs=[
            pl.BlockSpec((_DIM, 2 * _SEQ), lambda i: (0, 0)),
            pl.BlockSpec((1, _BI, _DIM, _SEQ), lambda i: (0, i, 0, 0)),
        ],
        out_specs=pl.BlockSpec((1, _BI, _DIM, _SEQ), lambda i: (0, i, 0, 0)),
        out_shape=jax.ShapeDtypeStruct(xt.shape, x.dtype),
    )(rt_p, xt)
    return jnp.transpose(out, (0, 1, 3, 2))
